# baseline (speedup 1.0000x reference)
import functools

import jax
import jax.numpy as jnp
from jax import lax
from jax.experimental import pallas as pl
from jax.experimental.pallas import tpu as pltpu
from jax.experimental.pallas import tpu_sc as plsc

_D_BYTE = 256
_D_FAM = 128
_D_MIC = 128
_DIM = _D_BYTE + _D_FAM + _D_MIC
_BATCH = 4
_SEQ = 8192
_B_TOTAL = _BATCH * _SEQ

_NC = 2
_NS = 16
_NW = _NC * _NS
_B_PER_W = _B_TOTAL // _NW
_CHUNK = 32
_N_CHUNKS = _B_PER_W // _CHUNK
_NBUF = 3
_L = 16

_mesh = plsc.VectorSubcoreMesh(core_axis_name="c", subcore_axis_name="s")


@functools.partial(
    pl.kernel,
    mesh=_mesh,
    out_type=jax.ShapeDtypeStruct((_B_TOTAL, _DIM), jnp.float32),
    scratch_types=[
        pltpu.VMEM((_B_PER_W,), jnp.int32),
        pltpu.VMEM((256, _D_BYTE), jnp.float32),
        pltpu.VMEM((4, _D_FAM), jnp.float32),
        pltpu.VMEM((64, _D_MIC), jnp.float32),
        pltpu.VMEM((_NBUF * _CHUNK, _DIM), jnp.float32),
        pltpu.SemaphoreType.DMA,
    ],
)
def _lookup_concat(packed_hbm, bt_hbm, ft_hbm, mt_hbm, out_hbm,
                   idx_v, bt_v, ft_v, mt_v, rows, wsem):
    wid = lax.axis_index("s") * _NC + lax.axis_index("c")
    base0 = wid * _B_PER_W

    pltpu.sync_copy(bt_hbm, bt_v)
    pltpu.sync_copy(ft_hbm, ft_v)
    pltpu.sync_copy(mt_hbm, mt_v)
    pltpu.sync_copy(packed_hbm.at[pl.ds(base0, _B_PER_W)], idx_v)

    def fill_rows(i):
        slot = lax.rem(i, _NBUF) * _CHUNK

        def grp_body(g, carry):
            pvec = idx_v[pl.ds(i * _CHUNK + g * _L, _L)]
            rbv = pvec & 255
            rfv = (pvec >> 8) & 255
            rmv = (pvec >> 16) & 255
            rbs = [rbv[j] for j in range(_L)]
            rfs = [rfv[j] for j in range(_L)]
            rms = [rmv[j] for j in range(_L)]
            base_row = slot + g * _L

            @plsc.parallel_loop(0, _D_BYTE // _L, unroll=4)
            def _byte_k(k):
                off = k * _L
                for j in range(_L):
                    rows[base_row + j, pl.ds(off, _L)] = bt_v[rbs[j], pl.ds(off, _L)]

            @plsc.parallel_loop(0, _D_FAM // _L, unroll=4)
            def _fam_k(k):
                off = k * _L
                for j in range(_L):
                    rows[base_row + j, pl.ds(_D_BYTE + off, _L)] = ft_v[rfs[j], pl.ds(off, _L)]

            @plsc.parallel_loop(0, _D_MIC // _L, unroll=4)
            def _mic_k(k):
                off = k * _L
                for j in range(_L):
                    rows[base_row + j, pl.ds(_D_BYTE + _D_FAM + off, _L)] = mt_v[rms[j], pl.ds(off, _L)]

            return carry

        lax.fori_loop(0, _CHUNK // _L, grp_body, 0)

    def write_desc(i):
        slot = lax.rem(i, _NBUF) * _CHUNK
        return pltpu.make_async_copy(
            rows.at[pl.ds(slot, _CHUNK), :],
            out_hbm.at[pl.ds(base0 + i * _CHUNK, _CHUNK), :], wsem)

    def body(i, carry):
        @pl.when(i >= _NBUF)
        def _():
            write_desc(0).wait()
        fill_rows(i)
        write_desc(i).start()
        return carry

    lax.fori_loop(0, _N_CHUNKS, body, 0)

    for _ in range(_NBUF):
        write_desc(0).wait()


def kernel(input_ids, families, micro_refs, byte_table, family_table, micro_table):
    packed = (input_ids.astype(jnp.int32)
              | (families.astype(jnp.int32) << 8)
              | (micro_refs.astype(jnp.int32) << 16)).reshape(_B_TOTAL)
    out = _lookup_concat(packed, byte_table, family_table, micro_table)
    return out.reshape(_BATCH, _SEQ, _DIM)


# speedup vs baseline: 1.1255x; 1.1255x over previous
import functools

import jax
import jax.numpy as jnp
from jax import lax
from jax.experimental import pallas as pl
from jax.experimental.pallas import tpu as pltpu
from jax.experimental.pallas import tpu_sc as plsc

_D_BYTE = 256
_D_FAM = 128
_D_MIC = 128
_DIM = _D_BYTE + _D_FAM + _D_MIC
_BATCH = 4
_SEQ = 8192
_B_TOTAL = _BATCH * _SEQ

_NC = 2
_NS = 16
_NW = _NC * _NS
_B_PER_W = _B_TOTAL // _NW
_CHUNK = 32
_N_CHUNKS = _B_PER_W // _CHUNK
_NBUF = 3
_L = 16

_mesh = plsc.VectorSubcoreMesh(core_axis_name="c", subcore_axis_name="s")


@functools.partial(
    pl.kernel,
    mesh=_mesh,
    out_type=jax.ShapeDtypeStruct((_B_TOTAL, _DIM), jnp.float32),
    scratch_types=[
        pltpu.VMEM((_B_PER_W + _L,), jnp.int32),
        pltpu.VMEM((256, _D_BYTE), jnp.float32),
        pltpu.VMEM((4, _D_FAM), jnp.float32),
        pltpu.VMEM((64, _D_MIC), jnp.float32),
        pltpu.VMEM((_NBUF * _CHUNK, _DIM), jnp.float32),
        pltpu.SemaphoreType.DMA,
    ],
)
def _lookup_concat(packed_hbm, bt_hbm, ft_hbm, mt_hbm, out_hbm,
                   idx_v, bt_v, ft_v, mt_v, rows, wsem):
    wid = lax.axis_index("s") * _NC + lax.axis_index("c")
    base0 = wid * _B_PER_W

    pltpu.sync_copy(bt_hbm, bt_v)
    pltpu.sync_copy(ft_hbm, ft_v)
    pltpu.sync_copy(mt_hbm, mt_v)
    pltpu.sync_copy(packed_hbm.at[pl.ds(base0, _B_PER_W)], idx_v.at[pl.ds(0, _B_PER_W)])

    def fill_rows(i):
        slot = lax.rem(i, _NBUF) * _CHUNK

        @plsc.parallel_loop(0, _CHUNK, unroll=2)
        def _row(j):
            pvec = idx_v[pl.ds(i * _CHUNK + j, _L)]
            p = pvec[0]
            rb = p & 255
            rf = (p >> 8) & 255
            rm = (p >> 16) & 255
            row = slot + j
            for k in range(_D_BYTE // _L):
                rows[row, pl.ds(k * _L, _L)] = bt_v[rb, pl.ds(k * _L, _L)]
            for k in range(_D_FAM // _L):
                rows[row, pl.ds(_D_BYTE + k * _L, _L)] = ft_v[rf, pl.ds(k * _L, _L)]
            for k in range(_D_MIC // _L):
                rows[row, pl.ds(_D_BYTE + _D_FAM + k * _L, _L)] = mt_v[rm, pl.ds(k * _L, _L)]

    def write_desc(i):
        slot = lax.rem(i, _NBUF) * _CHUNK
        return pltpu.make_async_copy(
            rows.at[pl.ds(slot, _CHUNK), :],
            out_hbm.at[pl.ds(base0 + i * _CHUNK, _CHUNK), :], wsem)

    def body(i, carry):
        @pl.when(i >= _NBUF)
        def _():
            write_desc(0).wait()
        fill_rows(i)
        write_desc(i).start()
        return carry

    lax.fori_loop(0, _N_CHUNKS, body, 0)

    for _ in range(_NBUF):
        write_desc(0).wait()


def kernel(input_ids, families, micro_refs, byte_table, family_table, micro_table):
    packed = (input_ids.astype(jnp.int32)
              | (families.astype(jnp.int32) << 8)
              | (micro_refs.astype(jnp.int32) << 16)).reshape(_B_TOTAL)
    out = _lookup_concat(packed, byte_table, family_table, micro_table)
    return out.reshape(_BATCH, _SEQ, _DIM)


# speedup vs baseline: 1.1425x; 1.0151x over previous
import functools

import jax
import jax.numpy as jnp
from jax import lax
from jax.experimental import pallas as pl
from jax.experimental.pallas import tpu as pltpu
from jax.experimental.pallas import tpu_sc as plsc

_D_BYTE = 256
_D_FAM = 128
_D_MIC = 128
_DIM = _D_BYTE + _D_FAM + _D_MIC
_BATCH = 4
_SEQ = 8192
_B_TOTAL = _BATCH * _SEQ

_NC = 2
_NS = 16
_NW = _NC * _NS
_B_PER_W = _B_TOTAL // _NW
_CHUNK = 32
_N_CHUNKS = _B_PER_W // _CHUNK
_NBUF = 3
_L = 16

_mesh = plsc.VectorSubcoreMesh(core_axis_name="c", subcore_axis_name="s")


@functools.partial(
    pl.kernel,
    mesh=_mesh,
    out_type=jax.ShapeDtypeStruct((_B_TOTAL, _DIM), jnp.float32),
    scratch_types=[
        pltpu.VMEM((_B_PER_W + _L,), jnp.int32),
        pltpu.VMEM((256, _D_BYTE), jnp.float32),
        pltpu.VMEM((4, _D_FAM), jnp.float32),
        pltpu.VMEM((64, _D_MIC), jnp.float32),
        pltpu.VMEM((_NBUF * _CHUNK, _DIM), jnp.float32),
        pltpu.SemaphoreType.DMA,
    ],
)
def _lookup_concat(packed_hbm, bt_hbm, ft_hbm, mt_hbm, out_hbm,
                   idx_v, bt_v, ft_v, mt_v, rows, wsem):
    wid = lax.axis_index("s") * _NC + lax.axis_index("c")
    base0 = wid * _B_PER_W

    pltpu.sync_copy(bt_hbm, bt_v)
    pltpu.sync_copy(ft_hbm, ft_v)
    pltpu.sync_copy(mt_hbm, mt_v)
    pltpu.sync_copy(packed_hbm.at[pl.ds(base0, _B_PER_W)], idx_v.at[pl.ds(0, _B_PER_W)])

    def fill_rows(i):
        slot = lax.rem(i, _NBUF) * _CHUNK

        @plsc.parallel_loop(0, _CHUNK, unroll=4)
        def _row(j):
            pvec = idx_v[pl.ds(i * _CHUNK + j, _L)]
            p = pvec[0]
            rb = p & 255
            rf = (p >> 8) & 255
            rm = (p >> 16) & 255
            row = slot + j
            for k in range(_D_BYTE // _L):
                rows[row, pl.ds(k * _L, _L)] = bt_v[rb, pl.ds(k * _L, _L)]
            for k in range(_D_FAM // _L):
                rows[row, pl.ds(_D_BYTE + k * _L, _L)] = ft_v[rf, pl.ds(k * _L, _L)]
            for k in range(_D_MIC // _L):
                rows[row, pl.ds(_D_BYTE + _D_FAM + k * _L, _L)] = mt_v[rm, pl.ds(k * _L, _L)]

    def write_desc(i):
        slot = lax.rem(i, _NBUF) * _CHUNK
        return pltpu.make_async_copy(
            rows.at[pl.ds(slot, _CHUNK), :],
            out_hbm.at[pl.ds(base0 + i * _CHUNK, _CHUNK), :], wsem)

    def body(i, carry):
        @pl.when(i >= _NBUF)
        def _():
            write_desc(0).wait()
        fill_rows(i)
        write_desc(i).start()
        return carry

    lax.fori_loop(0, _N_CHUNKS, body, 0)

    for _ in range(_NBUF):
        write_desc(0).wait()


def kernel(input_ids, families, micro_refs, byte_table, family_table, micro_table):
    packed = (input_ids.astype(jnp.int32)
              | (families.astype(jnp.int32) << 8)
              | (micro_refs.astype(jnp.int32) << 16)).reshape(_B_TOTAL)
    out = _lookup_concat(packed, byte_table, family_table, micro_table)
    return out.reshape(_BATCH, _SEQ, _DIM)


# speedup vs baseline: 1.1574x; 1.0130x over previous
import functools

import jax
import jax.numpy as jnp
from jax import lax
from jax.experimental import pallas as pl
from jax.experimental.pallas import tpu as pltpu
from jax.experimental.pallas import tpu_sc as plsc

_D_BYTE = 256
_D_FAM = 128
_D_MIC = 128
_DIM = _D_BYTE + _D_FAM + _D_MIC
_BATCH = 4
_SEQ = 8192
_B_TOTAL = _BATCH * _SEQ

_NC = 2
_NS = 16
_NW = _NC * _NS
_B_PER_W = _B_TOTAL // _NW
_CHUNK = 32
_N_CHUNKS = _B_PER_W // _CHUNK
_NBUF = 3
_L = 16

_mesh = plsc.VectorSubcoreMesh(core_axis_name="c", subcore_axis_name="s")


@functools.partial(
    pl.kernel,
    mesh=_mesh,
    out_type=jax.ShapeDtypeStruct((_B_TOTAL, _DIM), jnp.float32),
    scratch_types=[
        pltpu.VMEM((_B_PER_W + _L,), jnp.int32),
        pltpu.VMEM((256, _D_BYTE), jnp.float32),
        pltpu.VMEM((4, _D_FAM), jnp.float32),
        pltpu.VMEM((64, _D_MIC), jnp.float32),
        pltpu.VMEM((_NBUF * _CHUNK, _DIM), jnp.float32),
        pltpu.SemaphoreType.DMA,
    ],
)
def _lookup_concat(packed_hbm, bt_hbm, ft_hbm, mt_hbm, out_hbm,
                   idx_v, bt_v, ft_v, mt_v, rows, wsem):
    wid = lax.axis_index("s") * _NC + lax.axis_index("c")
    base0 = wid * _B_PER_W

    pltpu.sync_copy(bt_hbm, bt_v)
    pltpu.sync_copy(ft_hbm, ft_v)
    pltpu.sync_copy(mt_hbm, mt_v)
    pltpu.sync_copy(packed_hbm.at[pl.ds(base0, _B_PER_W)], idx_v.at[pl.ds(0, _B_PER_W)])

    def fill_rows(i):
        slot = lax.rem(i, _NBUF) * _CHUNK

        @plsc.parallel_loop(0, _CHUNK, unroll=8)
        def _row(j):
            pvec = idx_v[pl.ds(i * _CHUNK + j, _L)]
            p = pvec[0]
            rb = p & 255
            rf = (p >> 8) & 255
            rm = (p >> 16) & 255
            row = slot + j
            for k in range(_D_BYTE // _L):
                rows[row, pl.ds(k * _L, _L)] = bt_v[rb, pl.ds(k * _L, _L)]
            for k in range(_D_FAM // _L):
                rows[row, pl.ds(_D_BYTE + k * _L, _L)] = ft_v[rf, pl.ds(k * _L, _L)]
            for k in range(_D_MIC // _L):
                rows[row, pl.ds(_D_BYTE + _D_FAM + k * _L, _L)] = mt_v[rm, pl.ds(k * _L, _L)]

    def write_desc(i):
        slot = lax.rem(i, _NBUF) * _CHUNK
        return pltpu.make_async_copy(
            rows.at[pl.ds(slot, _CHUNK), :],
            out_hbm.at[pl.ds(base0 + i * _CHUNK, _CHUNK), :], wsem)

    def body(i, carry):
        @pl.when(i >= _NBUF)
        def _():
            write_desc(0).wait()
        fill_rows(i)
        write_desc(i).start()
        return carry

    lax.fori_loop(0, _N_CHUNKS, body, 0)

    for _ in range(_NBUF):
        write_desc(0).wait()


def kernel(input_ids, families, micro_refs, byte_table, family_table, micro_table):
    packed = (input_ids.astype(jnp.int32)
              | (families.astype(jnp.int32) << 8)
              | (micro_refs.astype(jnp.int32) << 16)).reshape(_B_TOTAL)
    out = _lookup_concat(packed, byte_table, family_table, micro_table)
    return out.reshape(_BATCH, _SEQ, _DIM)
